# Initial kernel scaffold; baseline (speedup 1.0000x reference)
import jax
import jax.numpy as jnp
from jax.experimental import pallas as pl


def kernel(x, edge_index, W0, b0, W1, b1):
    raise NotImplementedError("write your pallas kernel here")


# speedup vs baseline: 5.0042x; 5.0042x over previous
import functools

import jax
import jax.numpy as jnp
from jax import lax
from jax.experimental import pallas as pl
from jax.experimental.pallas import tpu as pltpu
from jax.experimental.pallas import tpu_sc as plsc

N = 10000
NPAD = 10240
E = 320000
D = 128
CC = 64

NC = 2
NS = 16
NW = NC * NS
EW = E // NW
K = 80
NCHUNK = EW // K
ROWS_PER_TILE = NPAD // NS

_mesh = plsc.VectorSubcoreMesh(core_axis_name="c", subcore_axis_name="s")


def _fill_1d(ref, n, val):
    def body(i, carry):
        ref[pl.ds(i * 16, 16)] = jnp.full((16,), val, jnp.float32)
        return carry
    lax.fori_loop(0, n // 16, body, 0)


@functools.partial(
    pl.kernel,
    out_type=jax.ShapeDtypeStruct((NC, 2, NPAD), jnp.float32),
    mesh=_mesh,
    scratch_types=[
        pltpu.VMEM((K,), jnp.int32),
        pltpu.VMEM((K,), jnp.float32),
        pltpu.VMEM((ROWS_PER_TILE,), jnp.float32),
        pltpu.VMEM_SHARED((NPAD,), jnp.float32),
        pltpu.VMEM_SHARED((NPAD,), jnp.float32),
    ],
    compiler_params=pltpu.CompilerParams(use_tc_tiling_on_sc=False),
)
def _sc_degrees(src_hbm, dst_hbm, deg_hbm, idx_v, ones_v, zeros_v,
                dout_sh, din_sh):
    c = lax.axis_index("c")
    s = lax.axis_index("s")
    _fill_1d(ones_v, K, 1.0)
    _fill_1d(zeros_v, ROWS_PER_TILE, 0.0)
    row0 = s * ROWS_PER_TILE
    pltpu.sync_copy(zeros_v, dout_sh.at[pl.ds(row0, ROWS_PER_TILE)])
    pltpu.sync_copy(zeros_v, din_sh.at[pl.ds(row0, ROWS_PER_TILE)])
    plsc.subcore_barrier()
    base = (c * NS + s) * EW

    def chunk(k, carry):
        off = base + k * K
        pltpu.sync_copy(src_hbm.at[pl.ds(off, K)], idx_v)
        pltpu.sync_copy(ones_v, dout_sh.at[idx_v], add=True)
        pltpu.sync_copy(dst_hbm.at[pl.ds(off, K)], idx_v)
        pltpu.sync_copy(ones_v, din_sh.at[idx_v], add=True)
        return carry

    lax.fori_loop(0, NCHUNK, chunk, 0)
    plsc.subcore_barrier()
    pltpu.sync_copy(dout_sh.at[pl.ds(row0, ROWS_PER_TILE)],
                    deg_hbm.at[c, 0, pl.ds(row0, ROWS_PER_TILE)])
    pltpu.sync_copy(din_sh.at[pl.ds(row0, ROWS_PER_TILE)],
                    deg_hbm.at[c, 1, pl.ds(row0, ROWS_PER_TILE)])


ZB = 64


@functools.partial(
    pl.kernel,
    out_type=jax.ShapeDtypeStruct((NC, NPAD, CC), jnp.float32),
    mesh=_mesh,
    scratch_types=[
        pltpu.VMEM((K,), jnp.int32),
        pltpu.VMEM((K,), jnp.int32),
        pltpu.VMEM((K, CC), jnp.float32),
        pltpu.VMEM((ZB, CC), jnp.float32),
        pltpu.VMEM_SHARED((NPAD, CC), jnp.float32),
        pltpu.SemaphoreType.DMA,
    ],
    compiler_params=pltpu.CompilerParams(use_tc_tiling_on_sc=False),
)
def _sc_segsum(z_hbm, src_hbm, dst_hbm, out_hbm, isrc_v, idst_v, rows_v,
               zb_v, agg_sh, sem):
    c = lax.axis_index("c")
    s = lax.axis_index("s")

    def zrow(i, carry):
        r = i // (CC // 16)
        j = i % (CC // 16)
        zb_v[r, pl.ds(j * 16, 16)] = jnp.zeros((16,), jnp.float32)
        return carry

    lax.fori_loop(0, ZB * (CC // 16), zrow, 0)
    row0 = s * ROWS_PER_TILE

    def zcopy(j, carry):
        pltpu.sync_copy(zb_v, agg_sh.at[pl.ds(row0 + j * ZB, ZB)])
        return carry

    lax.fori_loop(0, ROWS_PER_TILE // ZB, zcopy, 0)
    plsc.subcore_barrier()
    base = (c * NS + s) * EW

    def chunk(k, carry):
        off = base + k * K
        pltpu.sync_copy(src_hbm.at[pl.ds(off, K)], isrc_v)
        pltpu.sync_copy(dst_hbm.at[pl.ds(off, K)], idst_v)
        pltpu.async_copy(z_hbm.at[isrc_v], rows_v, sem).wait()
        pltpu.sync_copy(rows_v, agg_sh.at[idst_v], add=True)
        return carry

    lax.fori_loop(0, NCHUNK, chunk, 0)
    plsc.subcore_barrier()
    pltpu.sync_copy(agg_sh.at[pl.ds(row0, ROWS_PER_TILE)],
                    out_hbm.at[c, pl.ds(row0, ROWS_PER_TILE)])


def _tc_norms_body(deg_ref, x1_ref, ns_ref, nd_ref, z1_ref):
    deg = deg_ref[...]
    d = deg[0] + deg[1]
    do = d[0]
    di = d[1]
    ns = jnp.where(do > 0, lax.rsqrt(jnp.maximum(do, 1.0)), 0.0)
    nd = jnp.where(di > 0, lax.rsqrt(jnp.maximum(di, 1.0)), 0.0)
    ns_ref[...] = ns
    nd_ref[...] = nd
    z1_ref[...] = x1_ref[...] * ns


_tc_norms = pl.pallas_call(
    _tc_norms_body,
    out_shape=(
        jax.ShapeDtypeStruct((NPAD, 1), jnp.float32),
        jax.ShapeDtypeStruct((NPAD, 1), jnp.float32),
        jax.ShapeDtypeStruct((NPAD, CC), jnp.float32),
    ),
)


def _tc_ep1_body(p_ref, nd_ref, ns_ref, x0_ref, w_ref, b_ref, y0_ref, z2_ref):
    agg = (p_ref[0] + p_ref[1]) * nd_ref[...]
    y0 = (x0_ref[...] + b_ref[...][None, :]
          + jnp.dot(agg, w_ref[...], preferred_element_type=jnp.float32))
    y0_ref[...] = y0
    z2_ref[...] = y0 * ns_ref[...]


_tc_ep1 = pl.pallas_call(
    _tc_ep1_body,
    out_shape=(
        jax.ShapeDtypeStruct((NPAD, CC), jnp.float32),
        jax.ShapeDtypeStruct((NPAD, CC), jnp.float32),
    ),
)


def _tc_ep2_body(p_ref, nd_ref, x1_ref, y0_ref, w_ref, b_ref, out_ref):
    agg = (p_ref[0] + p_ref[1]) * nd_ref[...]
    y1 = (x1_ref[...] + b_ref[...][None, :]
          + jnp.dot(agg, w_ref[...], preferred_element_type=jnp.float32))
    out_ref[:, :CC] = y0_ref[...]
    out_ref[:, CC:] = y1


_tc_ep2 = pl.pallas_call(
    _tc_ep2_body,
    out_shape=jax.ShapeDtypeStruct((NPAD, D), jnp.float32),
)


def kernel(x, edge_index, W0, b0, W1, b1):
    src = edge_index[0].astype(jnp.int32)
    dst = edge_index[1].astype(jnp.int32)
    xp = jnp.pad(x, ((0, NPAD - N), (0, 0)))
    x0 = xp[:, :CC]
    x1 = xp[:, CC:]
    deg = _sc_degrees(src, dst).reshape(NC, 2, NPAD, 1)
    ns, nd, z1 = _tc_norms(deg, x1)
    p0 = _sc_segsum(z1, src, dst)
    y0, z2 = _tc_ep1(p0, nd, ns, x0, W0, b0)
    p1 = _sc_segsum(z2, src, dst)
    out = _tc_ep2(p1, nd, x1, y0, W1, b1)
    return out[:N]
